# baseline (speedup 1.0000x reference)
import jax
import jax.numpy as jnp
from jax.experimental import pallas as pl
from jax.experimental.pallas import tpu as pltpu


_ROWS = 16384
_COLS = 2048
_BLK = 512


def _concat_kernel(left_hbm, right_hbm, out_ref, sem_l, sem_r):
    i = pl.program_id(0)
    rows = pl.ds(i * _BLK, _BLK)
    cl = pltpu.make_async_copy(left_hbm.at[rows, :], out_ref.at[:, :_COLS], sem_l)
    cr = pltpu.make_async_copy(right_hbm.at[rows, :], out_ref.at[:, _COLS:], sem_r)
    cl.start()
    cr.start()
    cl.wait()
    cr.wait()


def kernel(left, right):
    n_blk = _ROWS // _BLK
    return pl.pallas_call(
        _concat_kernel,
        grid=(n_blk,),
        in_specs=[
            pl.BlockSpec(memory_space=pl.ANY),
            pl.BlockSpec(memory_space=pl.ANY),
        ],
        out_specs=pl.BlockSpec((_BLK, 2 * _COLS), lambda i: (i, 0)),
        out_shape=jax.ShapeDtypeStruct((_ROWS, 2 * _COLS), jnp.float32),
        scratch_shapes=[pltpu.SemaphoreType.DMA, pltpu.SemaphoreType.DMA],
    )(left, right)


# speedup vs baseline: 1.2455x; 1.2455x over previous
import jax
import jax.numpy as jnp
from jax.experimental import pallas as pl


_ROWS = 16384
_COLS = 2048
_BLK = 512


def _concat_kernel(left_ref, right_ref, out_ref):
    out_ref[:, :_COLS] = left_ref[:]
    out_ref[:, _COLS:] = right_ref[:]


def kernel(left, right):
    n_blk = _ROWS // _BLK
    return pl.pallas_call(
        _concat_kernel,
        grid=(n_blk,),
        in_specs=[
            pl.BlockSpec((_BLK, _COLS), lambda i: (i, 0)),
            pl.BlockSpec((_BLK, _COLS), lambda i: (i, 0)),
        ],
        out_specs=pl.BlockSpec((_BLK, 2 * _COLS), lambda i: (i, 0)),
        out_shape=jax.ShapeDtypeStruct((_ROWS, 2 * _COLS), jnp.float32),
    )(left, right)
